# baseline (speedup 1.0000x reference)
import functools

import jax
import jax.numpy as jnp
from jax.experimental import pallas as pl
from jax.experimental.pallas import tpu as pltpu
from jax.experimental.pallas import tpu_sc as plsc

_TILE = 64


def _router_kernel(xf_ref, rw_ref, ei_ref, scale_ref):
    f32 = jnp.float32
    xf = xf_ref[...]
    rw = rw_ref[...]
    T = xf.shape[0]
    E = rw.shape[0]

    logits = jax.lax.dot_general(
        xf, rw, (((1,), (1,)), ((), ())), preferred_element_type=f32)
    lmax = jnp.max(logits, axis=1, keepdims=True)
    ex = jnp.exp(logits - lmax)
    p = ex / jnp.sum(ex, axis=1, keepdims=True)

    lane = jax.lax.broadcasted_iota(jnp.int32, (T, E), 1)
    m1 = jnp.max(p, axis=1, keepdims=True)
    i1 = jnp.min(jnp.where(p == m1, lane, E), axis=1, keepdims=True)
    o1 = (lane == i1)
    pm = jnp.where(o1, -jnp.inf, p)
    m2 = jnp.max(pm, axis=1, keepdims=True)
    i2 = jnp.min(jnp.where(pm == m2, lane, E), axis=1, keepdims=True)

    ssum = m1 + m2
    ei_ref[...] = jnp.concatenate([i1, i2], axis=1)
    scale_ref[...] = jnp.concatenate([m1 / ssum, m2 / ssum], axis=1)


def _sc_group_body(ei_hbm, dest_hbm, te_hbm, na_hbm,
                   eis, dests, cnts, offs, tes, nas, sem,
                   *, n_assign, n_experts, n_tiles):
    idx = jax.lax.axis_index("core")

    @pl.when(idx == 0)
    def _():
        pltpu.async_copy(ei_hbm, eis, sem).wait()

        @pl.loop(0, n_experts)
        def _(e):
            cnts[e] = 0

        @pl.loop(0, n_assign)
        def _(a):
            e = eis[a]
            dests[a] = cnts[e]
            cnts[e] = cnts[e] + 1

        offs[0] = 0

        @pl.loop(0, n_experts)
        def _(e):
            pc = ((cnts[e] + (_TILE - 1)) // _TILE) * _TILE
            offs[e + 1] = offs[e] + pc

        nas[0] = 0

        @pl.loop(0, n_experts)
        def _(e):
            @pl.when(cnts[e] > 0)
            def _():
                nas[0] = e

        @pl.loop(0, n_tiles)
        def _(t):
            tes[t] = nas[0]

        @pl.loop(0, n_assign)
        def _(a):
            e = eis[a]
            d = offs[e] + dests[a]
            dests[a] = d
            tes[d // _TILE] = e

        nas[0] = offs[n_experts] // _TILE

        pltpu.async_copy(dests, dest_hbm, sem).wait()
        pltpu.async_copy(tes, te_hbm, sem).wait()
        pltpu.async_copy(nas, na_hbm, sem).wait()


def _moe_kernel(te_ref, na_ref, xf_ref, w1_ref, w2_ref, w3_ref,
                dest_ref, scale_ref, out_ref):
    f32 = jnp.float32
    t = pl.program_id(0)
    T = xf_ref.shape[0]

    @pl.when(t == 0)
    def _():
        out_ref[...] = jnp.zeros_like(out_ref)

    @pl.when(t < na_ref[0])
    def _():
        d0 = dest_ref[:, 0:1]
        d1 = dest_ref[:, 1:2]
        s0 = scale_ref[:, 0:1]
        s1 = scale_ref[:, 1:2]
        rows = t * _TILE + jax.lax.broadcasted_iota(jnp.int32, (T, _TILE), 1)
        m0 = (rows == d0)
        m1 = (rows == d1)
        g = m0.astype(f32) + m1.astype(f32)
        gs = (m0.astype(f32) * s0 + m1.astype(f32) * s1)

        dotg = functools.partial(jax.lax.dot_general,
                                 preferred_element_type=f32)
        xg = dotg(g, xf_ref[...], (((0,), (0,)), ((), ())))
        w1 = w1_ref[0]
        w3 = w3_ref[0]
        w2 = w2_ref[0]
        h1 = dotg(xg, w1, (((1,), (1,)), ((), ())))
        h3 = dotg(xg, w3, (((1,), (1,)), ((), ())))
        h = jax.nn.silu(h1) * h3
        y = dotg(h, w2, (((1,), (1,)), ((), ())))
        out_ref[...] += dotg(gs, y, (((1,), (0,)), ((), ())))


def kernel(x, router_w, w1, w2, w3):
    orig_shape = x.shape
    H = x.shape[-1]
    xf = x.reshape(-1, H)
    T = xf.shape[0]
    E = router_w.shape[0]
    EXP = w1.shape[1]
    n_tiles = (2 * T) // _TILE + E
    n_assign = 2 * T

    ei, scale = pl.pallas_call(
        _router_kernel,
        out_shape=(
            jax.ShapeDtypeStruct((T, 2), jnp.int32),
            jax.ShapeDtypeStruct((T, 2), jnp.float32),
        ),
    )(xf, router_w)

    sc_mesh = plsc.ScalarSubcoreMesh(axis_name="core", num_cores=2)
    sc_group = pl.kernel(
        functools.partial(_sc_group_body, n_assign=n_assign,
                          n_experts=E, n_tiles=n_tiles),
        out_type=(
            jax.ShapeDtypeStruct((n_assign,), jnp.int32),
            jax.ShapeDtypeStruct((n_tiles,), jnp.int32),
            jax.ShapeDtypeStruct((1,), jnp.int32),
        ),
        mesh=sc_mesh,
        scratch_types=[
            pltpu.SMEM((n_assign,), jnp.int32),
            pltpu.SMEM((n_assign,), jnp.int32),
            pltpu.SMEM((E,), jnp.int32),
            pltpu.SMEM((E + 1,), jnp.int32),
            pltpu.SMEM((n_tiles,), jnp.int32),
            pltpu.SMEM((1,), jnp.int32),
            pltpu.SemaphoreType.DMA,
        ],
    )
    dest_flat, te, na = sc_group(ei.reshape(n_assign))
    dest = dest_flat.reshape(T, 2)

    grid_spec = pltpu.PrefetchScalarGridSpec(
        num_scalar_prefetch=2,
        grid=(n_tiles,),
        in_specs=[
            pl.BlockSpec((T, H), lambda i, te, na: (0, 0)),
            pl.BlockSpec((1, EXP, H), lambda i, te, na: (te[i], 0, 0)),
            pl.BlockSpec((1, H, EXP), lambda i, te, na: (te[i], 0, 0)),
            pl.BlockSpec((1, EXP, H), lambda i, te, na: (te[i], 0, 0)),
            pl.BlockSpec((T, 2), lambda i, te, na: (0, 0)),
            pl.BlockSpec((T, 2), lambda i, te, na: (0, 0)),
        ],
        out_specs=pl.BlockSpec((T, H), lambda i, te, na: (0, 0)),
    )
    out = pl.pallas_call(
        _moe_kernel,
        grid_spec=grid_spec,
        out_shape=jax.ShapeDtypeStruct((T, H), jnp.float32),
        compiler_params=pltpu.CompilerParams(
            dimension_semantics=("arbitrary",)),
    )(te, na, xf, w1, w2, w3, dest, scale)

    return out.reshape(orig_shape)


# speedup vs baseline: 1.1328x; 1.1328x over previous
import functools

import jax
import jax.numpy as jnp
from jax.experimental import pallas as pl
from jax.experimental.pallas import tpu as pltpu

_TILE = 64


def _router_kernel(xf_ref, rw_ref, dest_ref, scale_ref, te_ref, na_ref,
                   *, n_tiles):
    f32 = jnp.float32
    xf = xf_ref[...]
    rw = rw_ref[...]
    T = xf.shape[0]
    E = rw.shape[0]

    logits = jax.lax.dot_general(
        xf, rw, (((1,), (1,)), ((), ())), preferred_element_type=f32)
    lmax = jnp.max(logits, axis=1, keepdims=True)
    ex = jnp.exp(logits - lmax)
    p = ex / jnp.sum(ex, axis=1, keepdims=True)

    lane = jax.lax.broadcasted_iota(jnp.int32, (T, E), 1)
    m1 = jnp.max(p, axis=1, keepdims=True)
    i1 = jnp.min(jnp.where(p == m1, lane, E), axis=1, keepdims=True)
    o1 = (lane == i1)
    pm = jnp.where(o1, -jnp.inf, p)
    m2 = jnp.max(pm, axis=1, keepdims=True)
    i2 = jnp.min(jnp.where(pm == m2, lane, E), axis=1, keepdims=True)
    o2 = (lane == i2)
    o1f = o1.astype(f32)
    o2f = o2.astype(f32)

    ssum = m1 + m2
    s0 = m1 / ssum
    s1 = m2 / ssum

    ones_t = jnp.ones((T, 1), f32)
    dotg = functools.partial(jax.lax.dot_general, preferred_element_type=f32)
    cnt1_c = dotg(o1f, ones_t, (((0,), (0,)), ((), ())))
    cnt_c = cnt1_c + dotg(o2f, ones_t, (((0,), (0,)), ((), ())))
    cnt_i = cnt_c.astype(jnp.int32)
    pc_i = ((cnt_i + (_TILE - 1)) // _TILE) * _TILE
    pcf = pc_i.astype(f32)

    er = jax.lax.broadcasted_iota(jnp.int32, (E, E), 0)
    ec = jax.lax.broadcasted_iota(jnp.int32, (E, E), 1)
    ls_e = (ec < er).astype(f32)
    off_c = dotg(ls_e, pcf, (((1,), (0,)), ((), ())))

    tr = jax.lax.broadcasted_iota(jnp.int32, (T, T), 0)
    tc = jax.lax.broadcasted_iota(jnp.int32, (T, T), 1)
    ls_t = (tc < tr).astype(f32)
    c1 = dotg(ls_t, o1f, (((1,), (0,)), ((), ())))
    rank0 = jnp.sum(o1f * c1, axis=1, keepdims=True)
    c2 = dotg(ls_t, o2f, (((1,), (0,)), ((), ())))
    rank1 = (jnp.sum(o2f * c2, axis=1, keepdims=True)
             + dotg(o2f, cnt1_c, (((1,), (0,)), ((), ()))))

    off0 = dotg(o1f, off_c, (((1,), (0,)), ((), ())))
    off1 = dotg(o2f, off_c, (((1,), (0,)), ((), ())))
    dest0 = (off0 + rank0).astype(jnp.int32)
    dest1 = (off1 + rank1).astype(jnp.int32)
    dest_ref[...] = jnp.concatenate([dest0, dest1], axis=1)
    scale_ref[...] = jnp.concatenate([s0, s1], axis=1)

    ones_e = jnp.ones((E, 1), f32)
    tot = dotg(pcf, ones_e, (((0,), (0,)), ((), ())))
    tot_i = tot.astype(jnp.int32)
    ends_i = (off_c + pcf).astype(jnp.int32)
    tstart = jax.lax.broadcasted_iota(jnp.int32, (E, n_tiles), 1) * _TILE
    num_le = jnp.sum((ends_i <= tstart).astype(jnp.int32), axis=0,
                     keepdims=True)
    te_act = jnp.minimum(num_le, E - 1)
    e_iota = jax.lax.broadcasted_iota(jnp.int32, (E, 1), 0)
    last_e = jnp.max(jnp.where(cnt_i > 0, e_iota, 0), axis=0, keepdims=True)
    tile_i = jax.lax.broadcasted_iota(jnp.int32, (1, n_tiles), 1)
    active = (tile_i * _TILE) < tot_i
    te_ref[...] = jnp.where(active, te_act, last_e)
    na_ref[...] = tot_i // _TILE


def _moe_kernel(te_ref, na_ref, xf_ref, w1_ref, w2_ref, w3_ref,
                dest_ref, scale_ref, out_ref):
    f32 = jnp.float32
    t = pl.program_id(0)
    T = xf_ref.shape[0]

    @pl.when(t == 0)
    def _():
        out_ref[...] = jnp.zeros_like(out_ref)

    @pl.when(t < na_ref[0])
    def _():
        d0 = dest_ref[:, 0:1]
        d1 = dest_ref[:, 1:2]
        s0 = scale_ref[:, 0:1]
        s1 = scale_ref[:, 1:2]
        rows = t * _TILE + jax.lax.broadcasted_iota(jnp.int32, (T, _TILE), 1)
        m0 = (rows == d0)
        m1 = (rows == d1)
        g = m0.astype(f32) + m1.astype(f32)
        gs = (m0.astype(f32) * s0 + m1.astype(f32) * s1)

        dotg = functools.partial(jax.lax.dot_general,
                                 preferred_element_type=f32)
        xg = dotg(g, xf_ref[...], (((0,), (0,)), ((), ())))
        w1 = w1_ref[0]
        w3 = w3_ref[0]
        w2 = w2_ref[0]
        h1 = dotg(xg, w1, (((1,), (1,)), ((), ())))
        h3 = dotg(xg, w3, (((1,), (1,)), ((), ())))
        h = jax.nn.silu(h1) * h3
        y = dotg(h, w2, (((1,), (1,)), ((), ())))
        out_ref[...] += dotg(gs, y, (((1,), (0,)), ((), ())))


def kernel(x, router_w, w1, w2, w3):
    orig_shape = x.shape
    H = x.shape[-1]
    xf = x.reshape(-1, H)
    T = xf.shape[0]
    E = router_w.shape[0]
    EXP = w1.shape[1]
    n_tiles = (2 * T) // _TILE + E

    dest, scale, te, na = pl.pallas_call(
        functools.partial(_router_kernel, n_tiles=n_tiles),
        out_shape=(
            jax.ShapeDtypeStruct((T, 2), jnp.int32),
            jax.ShapeDtypeStruct((T, 2), jnp.float32),
            jax.ShapeDtypeStruct((1, n_tiles), jnp.int32),
            jax.ShapeDtypeStruct((1, 1), jnp.int32),
        ),
    )(xf, router_w)

    grid_spec = pltpu.PrefetchScalarGridSpec(
        num_scalar_prefetch=2,
        grid=(n_tiles,),
        in_specs=[
            pl.BlockSpec((T, H), lambda i, te, na: (0, 0)),
            pl.BlockSpec((1, EXP, H), lambda i, te, na: (te[i], 0, 0)),
            pl.BlockSpec((1, H, EXP), lambda i, te, na: (te[i], 0, 0)),
            pl.BlockSpec((1, EXP, H), lambda i, te, na: (te[i], 0, 0)),
            pl.BlockSpec((T, 2), lambda i, te, na: (0, 0)),
            pl.BlockSpec((T, 2), lambda i, te, na: (0, 0)),
        ],
        out_specs=pl.BlockSpec((T, H), lambda i, te, na: (0, 0)),
    )
    out = pl.pallas_call(
        _moe_kernel,
        grid_spec=grid_spec,
        out_shape=jax.ShapeDtypeStruct((T, H), jnp.float32),
        compiler_params=pltpu.CompilerParams(
            dimension_semantics=("arbitrary",)),
    )(te.reshape(n_tiles), na.reshape(1), xf, w1, w2, w3, dest, scale)

    return out.reshape(orig_shape)
